# baseline (speedup 1.0000x reference)
import functools
import math

import jax
import jax.numpy as jnp
from jax.experimental import pallas as pl

_MAX_FREQUENCY = 128.0
_NUM_TOTAL_FREQUENCIES = 128
_BN = 512


def _pp2d_block(x_ref, pos_ref, scl_ref, rot_ref, coef_ref, freq_ref,
                col_ref, out_ref):
    xb = x_ref[...]
    x0 = xb[:, 0:1]
    x1 = xb[:, 1:2]
    pos = pos_ref[...]
    scl = scl_ref[...]
    rot = rot_ref[...]
    c = jnp.cos(rot)
    s = jnp.sin(rot)
    dx = x0 - pos[0:1, :]
    dy = x1 - pos[1:2, :]
    tx = c * dx + s * dy
    ty = c * dy - s * dx
    gx = tx * scl[0:1, :]
    gy = ty * scl[1:2, :]
    env = jnp.exp(-0.5 * (gx * gx + gy * gy))
    wave = jnp.zeros_like(tx)
    for k in range(freq_ref.shape[0]):
        fk = freq_ref[k:k + 1, :]
        ck = coef_ref[k:k + 1, :]
        wave = wave + ck * jnp.cos((2.0 * math.pi) * fk * tx)
    vals = env * wave
    out_ref[...] = jnp.dot(vals, col_ref[...],
                           preferred_element_type=jnp.float32)


def kernel(x, gaussian_colors, gaussian_positions, gaussian_scales,
           gaussian_rotations, topk_wave_coefficients, topk_wave_indices):
    n, _ = x.shape
    g, num_out = gaussian_colors.shape
    k = topk_wave_coefficients.shape[1]
    freqs = (topk_wave_indices.astype(jnp.float32)
             * (_MAX_FREQUENCY / _NUM_TOTAL_FREQUENCIES)).T
    coefs = topk_wave_coefficients.T
    pos_t = gaussian_positions.T
    scl_t = gaussian_scales.T
    rot_t = gaussian_rotations.T

    grid = (n // _BN,)
    out = pl.pallas_call(
        _pp2d_block,
        grid=grid,
        in_specs=[
            pl.BlockSpec((_BN, 2), lambda i: (i, 0)),
            pl.BlockSpec((2, g), lambda i: (0, 0)),
            pl.BlockSpec((2, g), lambda i: (0, 0)),
            pl.BlockSpec((1, g), lambda i: (0, 0)),
            pl.BlockSpec((k, g), lambda i: (0, 0)),
            pl.BlockSpec((k, g), lambda i: (0, 0)),
            pl.BlockSpec((g, num_out), lambda i: (0, 0)),
        ],
        out_specs=pl.BlockSpec((_BN, num_out), lambda i: (i, 0)),
        out_shape=jax.ShapeDtypeStruct((n, num_out), jnp.float32),
    )(x, pos_t, scl_t, rot_t, coefs, freqs, gaussian_colors)
    return out


# speedup vs baseline: 5.2186x; 5.2186x over previous
import functools
import math

import jax
import jax.numpy as jnp
from jax.experimental import pallas as pl

_MAX_FREQUENCY = 128.0
_NUM_TOTAL_FREQUENCIES = 128
_BN = 512

_COS_C = (0.9999994436793999, -19.739034372931183, 64.93061336990594,
          -85.29597096155109, 58.91255532445823, -21.283021593055757)
_EXP_C = (0.9999970019188498, -0.49995715187165096, 0.12485142322238095,
          -0.020618982395633076, 0.0024480984392707544,
          -0.00019859206665905042, 8.253586714950582e-06)


def _cos2pi_frac(p):
    u = p - jnp.round(p)
    t = u * u
    acc = jnp.float32(_COS_C[-1])
    for c in _COS_C[-2::-1]:
        acc = acc * t + jnp.float32(c)
    return acc


def _exp_neg_half(r):
    acc = jnp.float32(_EXP_C[-1])
    for c in _EXP_C[-2::-1]:
        acc = acc * r + jnp.float32(c)
    return acc


def _pp2d_block(x_ref, pos_ref, scl_ref, rot_ref, coef_ref, freq_ref,
                col_ref, out_ref):
    xb = x_ref[...]
    x0 = xb[:, 0:1]
    x1 = xb[:, 1:2]
    pos = pos_ref[...]
    scl = scl_ref[...]
    rot = rot_ref[...]
    c = jnp.cos(rot)
    s = jnp.sin(rot)
    dx = x0 - pos[0:1, :]
    dy = x1 - pos[1:2, :]
    tx = c * dx + s * dy
    ty = c * dy - s * dx
    gx = tx * scl[0:1, :]
    gy = ty * scl[1:2, :]
    env = _exp_neg_half(gx * gx + gy * gy)
    wave = jnp.zeros_like(tx)
    for k in range(freq_ref.shape[0]):
        fk = freq_ref[k:k + 1, :]
        ck = coef_ref[k:k + 1, :]
        wave = wave + ck * _cos2pi_frac(fk * tx)
    vals = env * wave
    out_ref[...] = jnp.dot(vals, col_ref[...],
                           preferred_element_type=jnp.float32)


def kernel(x, gaussian_colors, gaussian_positions, gaussian_scales,
           gaussian_rotations, topk_wave_coefficients, topk_wave_indices):
    n, _ = x.shape
    g, num_out = gaussian_colors.shape
    k = topk_wave_coefficients.shape[1]
    freqs = (topk_wave_indices.astype(jnp.float32)
             * (_MAX_FREQUENCY / _NUM_TOTAL_FREQUENCIES)).T
    coefs = topk_wave_coefficients.T
    pos_t = gaussian_positions.T
    scl_t = gaussian_scales.T
    rot_t = gaussian_rotations.T

    grid = (n // _BN,)
    out = pl.pallas_call(
        _pp2d_block,
        grid=grid,
        in_specs=[
            pl.BlockSpec((_BN, 2), lambda i: (i, 0)),
            pl.BlockSpec((2, g), lambda i: (0, 0)),
            pl.BlockSpec((2, g), lambda i: (0, 0)),
            pl.BlockSpec((1, g), lambda i: (0, 0)),
            pl.BlockSpec((k, g), lambda i: (0, 0)),
            pl.BlockSpec((k, g), lambda i: (0, 0)),
            pl.BlockSpec((g, num_out), lambda i: (0, 0)),
        ],
        out_specs=pl.BlockSpec((_BN, num_out), lambda i: (i, 0)),
        out_shape=jax.ShapeDtypeStruct((n, num_out), jnp.float32),
    )(x, pos_t, scl_t, rot_t, coefs, freqs, gaussian_colors)
    return out


# speedup vs baseline: 6.7486x; 1.2932x over previous
import jax
import jax.numpy as jnp
from jax.experimental import pallas as pl

_MAX_FREQUENCY = 128.0
_NUM_TOTAL_FREQUENCIES = 128
_BN = 512

_COS_C = (0.9989871016246259, -19.591096382371575, 61.5970720980049,
          -61.08884330070406)
_EXP_C = (0.9995561275689929, -0.49653966087404844, 0.11858208591144663,
          -0.016119124349784134, 0.0010024170403828251)


def _pp2d_block(x_ref, pos_ref, scl_ref, rot_ref, coef_ref, freq_ref,
                col_ref, out_ref):
    xb = x_ref[...]
    x0 = xb[:, 0:1]
    x1 = xb[:, 1:2]
    pos = pos_ref[...]
    scl = scl_ref[...]
    rot = rot_ref[...]
    c = jnp.cos(rot)
    s = jnp.sin(rot)
    sx = scl[0:1, :]
    sy = scl[1:2, :]
    v1 = -s * sy
    v2 = c * sy
    dx = x0 - pos[0:1, :]
    dy = x1 - pos[1:2, :]
    tx = c * dx + s * dy
    gx = tx * sx
    gy = v1 * dx + v2 * dy
    r2 = gx * gx + gy * gy
    env = jnp.float32(_EXP_C[-1])
    for a in _EXP_C[-2::-1]:
        env = env * r2 + jnp.float32(a)
    wave = None
    for k in range(freq_ref.shape[0]):
        fk = freq_ref[k:k + 1, :]
        ck = coef_ref[k:k + 1, :]
        p = fk * tx
        u = p - jnp.round(p)
        t = u * u
        acc = ck * jnp.float32(_COS_C[-1])
        for a in _COS_C[-2::-1]:
            acc = acc * t + ck * jnp.float32(a)
        wave = acc if wave is None else wave + acc
    out_ref[...] = jnp.dot(env * wave, col_ref[...],
                           preferred_element_type=jnp.float32)


def kernel(x, gaussian_colors, gaussian_positions, gaussian_scales,
           gaussian_rotations, topk_wave_coefficients, topk_wave_indices):
    n, _ = x.shape
    g, num_out = gaussian_colors.shape
    k = topk_wave_coefficients.shape[1]
    freqs = (topk_wave_indices.astype(jnp.float32)
             * (_MAX_FREQUENCY / _NUM_TOTAL_FREQUENCIES)).T
    coefs = topk_wave_coefficients.T
    pos_t = gaussian_positions.T
    scl_t = gaussian_scales.T
    rot_t = gaussian_rotations.T

    grid = (n // _BN,)
    out = pl.pallas_call(
        _pp2d_block,
        grid=grid,
        in_specs=[
            pl.BlockSpec((_BN, 2), lambda i: (i, 0)),
            pl.BlockSpec((2, g), lambda i: (0, 0)),
            pl.BlockSpec((2, g), lambda i: (0, 0)),
            pl.BlockSpec((1, g), lambda i: (0, 0)),
            pl.BlockSpec((k, g), lambda i: (0, 0)),
            pl.BlockSpec((k, g), lambda i: (0, 0)),
            pl.BlockSpec((g, num_out), lambda i: (0, 0)),
        ],
        out_specs=pl.BlockSpec((_BN, num_out), lambda i: (i, 0)),
        out_shape=jax.ShapeDtypeStruct((n, num_out), jnp.float32),
    )(x, pos_t, scl_t, rot_t, coefs, freqs, gaussian_colors)
    return out


# speedup vs baseline: 6.9392x; 1.0282x over previous
import jax
import jax.numpy as jnp
from jax.experimental import pallas as pl

_MAX_FREQUENCY = 128.0
_NUM_TOTAL_FREQUENCIES = 128
_BN = 1024

_COS_C = (0.9989871016246259, -19.591096382371575, 61.5970720980049,
          -61.08884330070406)
_EXP_C = (0.9995561275689929, -0.49653966087404844, 0.11858208591144663,
          -0.016119124349784134, 0.0010024170403828251)


def _pp2d_block(x_ref, pos_ref, scl_ref, rot_ref, coef_ref, freq_ref,
                col_ref, out_ref):
    xb = x_ref[...]
    x0 = xb[:, 0:1]
    x1 = xb[:, 1:2]
    pos = pos_ref[...]
    scl = scl_ref[...]
    rot = rot_ref[...]
    c = jnp.cos(rot)
    s = jnp.sin(rot)
    sx = scl[0:1, :]
    sy = scl[1:2, :]
    v1 = -s * sy
    v2 = c * sy
    dx = x0 - pos[0:1, :]
    dy = x1 - pos[1:2, :]
    tx = c * dx + s * dy
    gx = tx * sx
    gy = v1 * dx + v2 * dy
    r2 = gx * gx + gy * gy
    env = jnp.float32(_EXP_C[-1])
    for a in _EXP_C[-2::-1]:
        env = env * r2 + jnp.float32(a)
    wave = None
    for k in range(freq_ref.shape[0]):
        fk = freq_ref[k:k + 1, :]
        ck = coef_ref[k:k + 1, :]
        p = fk * tx
        u = p - jnp.round(p)
        t = u * u
        acc = ck * jnp.float32(_COS_C[-1])
        for a in _COS_C[-2::-1]:
            acc = acc * t + ck * jnp.float32(a)
        wave = acc if wave is None else wave + acc
    out_ref[...] = jnp.dot(env * wave, col_ref[...],
                           preferred_element_type=jnp.float32)


def kernel(x, gaussian_colors, gaussian_positions, gaussian_scales,
           gaussian_rotations, topk_wave_coefficients, topk_wave_indices):
    n, _ = x.shape
    g, num_out = gaussian_colors.shape
    k = topk_wave_coefficients.shape[1]
    freqs = (topk_wave_indices.astype(jnp.float32)
             * (_MAX_FREQUENCY / _NUM_TOTAL_FREQUENCIES)).T
    coefs = topk_wave_coefficients.T
    pos_t = gaussian_positions.T
    scl_t = gaussian_scales.T
    rot_t = gaussian_rotations.T

    grid = (n // _BN,)
    out = pl.pallas_call(
        _pp2d_block,
        grid=grid,
        in_specs=[
            pl.BlockSpec((_BN, 2), lambda i: (i, 0)),
            pl.BlockSpec((2, g), lambda i: (0, 0)),
            pl.BlockSpec((2, g), lambda i: (0, 0)),
            pl.BlockSpec((1, g), lambda i: (0, 0)),
            pl.BlockSpec((k, g), lambda i: (0, 0)),
            pl.BlockSpec((k, g), lambda i: (0, 0)),
            pl.BlockSpec((g, num_out), lambda i: (0, 0)),
        ],
        out_specs=pl.BlockSpec((_BN, num_out), lambda i: (i, 0)),
        out_shape=jax.ShapeDtypeStruct((n, num_out), jnp.float32),
    )(x, pos_t, scl_t, rot_t, coefs, freqs, gaussian_colors)
    return out


# speedup vs baseline: 6.9877x; 1.0070x over previous
import jax
import jax.numpy as jnp
from jax.experimental import pallas as pl

_MAX_FREQUENCY = 128.0
_NUM_TOTAL_FREQUENCIES = 128
_BN = 2048

_COS_C = (0.9989871016246259, -19.591096382371575, 61.5970720980049,
          -61.08884330070406)
_EXP_C = (0.9995561275689929, -0.49653966087404844, 0.11858208591144663,
          -0.016119124349784134, 0.0010024170403828251)


def _pp2d_block(x_ref, pos_ref, scl_ref, rot_ref, coef_ref, freq_ref,
                col_ref, out_ref):
    xb = x_ref[...]
    x0 = xb[:, 0:1]
    x1 = xb[:, 1:2]
    pos = pos_ref[...]
    scl = scl_ref[...]
    rot = rot_ref[...]
    c = jnp.cos(rot)
    s = jnp.sin(rot)
    sx = scl[0:1, :]
    sy = scl[1:2, :]
    v1 = -s * sy
    v2 = c * sy
    dx = x0 - pos[0:1, :]
    dy = x1 - pos[1:2, :]
    tx = c * dx + s * dy
    gx = tx * sx
    gy = v1 * dx + v2 * dy
    r2 = gx * gx + gy * gy
    env = jnp.float32(_EXP_C[-1])
    for a in _EXP_C[-2::-1]:
        env = env * r2 + jnp.float32(a)
    wave = None
    for k in range(freq_ref.shape[0]):
        fk = freq_ref[k:k + 1, :]
        ck = coef_ref[k:k + 1, :]
        p = fk * tx
        u = p - jnp.round(p)
        t = u * u
        acc = ck * jnp.float32(_COS_C[-1])
        for a in _COS_C[-2::-1]:
            acc = acc * t + ck * jnp.float32(a)
        wave = acc if wave is None else wave + acc
    out_ref[...] = jnp.dot(env * wave, col_ref[...],
                           preferred_element_type=jnp.float32)


def kernel(x, gaussian_colors, gaussian_positions, gaussian_scales,
           gaussian_rotations, topk_wave_coefficients, topk_wave_indices):
    n, _ = x.shape
    g, num_out = gaussian_colors.shape
    k = topk_wave_coefficients.shape[1]
    freqs = (topk_wave_indices.astype(jnp.float32)
             * (_MAX_FREQUENCY / _NUM_TOTAL_FREQUENCIES)).T
    coefs = topk_wave_coefficients.T
    pos_t = gaussian_positions.T
    scl_t = gaussian_scales.T
    rot_t = gaussian_rotations.T

    grid = (n // _BN,)
    out = pl.pallas_call(
        _pp2d_block,
        grid=grid,
        in_specs=[
            pl.BlockSpec((_BN, 2), lambda i: (i, 0)),
            pl.BlockSpec((2, g), lambda i: (0, 0)),
            pl.BlockSpec((2, g), lambda i: (0, 0)),
            pl.BlockSpec((1, g), lambda i: (0, 0)),
            pl.BlockSpec((k, g), lambda i: (0, 0)),
            pl.BlockSpec((k, g), lambda i: (0, 0)),
            pl.BlockSpec((g, num_out), lambda i: (0, 0)),
        ],
        out_specs=pl.BlockSpec((_BN, num_out), lambda i: (i, 0)),
        out_shape=jax.ShapeDtypeStruct((n, num_out), jnp.float32),
    )(x, pos_t, scl_t, rot_t, coefs, freqs, gaussian_colors)
    return out


# speedup vs baseline: 7.4616x; 1.0678x over previous
import jax
import jax.numpy as jnp
from jax.experimental import pallas as pl

_MAX_FREQUENCY = 128.0
_NUM_TOTAL_FREQUENCIES = 128
_BN = 2048

_COS_C = (0.9989871016246259, -19.591096382371575, 61.5970720980049,
          -61.08884330070406)
_EXP_C = (0.9995561275689929, -0.49653966087404844, 0.11858208591144663,
          -0.016119124349784134, 0.0010024170403828251)


def _pp2d_block(x_ref, pos_ref, scl_ref, rot_ref, coef_ref, freq_ref,
                col_ref, out_ref):
    xb = x_ref[...]
    x0 = xb[:, 0:1]
    x1 = xb[:, 1:2]
    pos = pos_ref[...]
    scl = scl_ref[...]
    rot = rot_ref[...]
    c = jnp.cos(rot)
    s = jnp.sin(rot)
    sx = scl[0:1, :]
    sy = scl[1:2, :]
    v1 = -s * sy
    v2 = c * sy
    dx = x0 - pos[0:1, :]
    dy = x1 - pos[1:2, :]
    tx = c * dx + s * dy
    gx = tx * sx
    gy = v1 * dx + v2 * dy
    r2 = gx * gx + gy * gy
    env = jnp.exp(-0.5 * r2)
    wave = None
    for k in range(freq_ref.shape[0]):
        fk = freq_ref[k:k + 1, :]
        ck = coef_ref[k:k + 1, :]
        p = fk * tx
        u = p - jnp.round(p)
        t = u * u
        acc = ck * jnp.float32(_COS_C[-1])
        for a in _COS_C[-2::-1]:
            acc = acc * t + ck * jnp.float32(a)
        wave = acc if wave is None else wave + acc
    out_ref[...] = jnp.dot(env * wave, col_ref[...],
                           preferred_element_type=jnp.float32)


def kernel(x, gaussian_colors, gaussian_positions, gaussian_scales,
           gaussian_rotations, topk_wave_coefficients, topk_wave_indices):
    n, _ = x.shape
    g, num_out = gaussian_colors.shape
    k = topk_wave_coefficients.shape[1]
    freqs = (topk_wave_indices.astype(jnp.float32)
             * (_MAX_FREQUENCY / _NUM_TOTAL_FREQUENCIES)).T
    coefs = topk_wave_coefficients.T
    pos_t = gaussian_positions.T
    scl_t = gaussian_scales.T
    rot_t = gaussian_rotations.T

    grid = (n // _BN,)
    out = pl.pallas_call(
        _pp2d_block,
        grid=grid,
        in_specs=[
            pl.BlockSpec((_BN, 2), lambda i: (i, 0)),
            pl.BlockSpec((2, g), lambda i: (0, 0)),
            pl.BlockSpec((2, g), lambda i: (0, 0)),
            pl.BlockSpec((1, g), lambda i: (0, 0)),
            pl.BlockSpec((k, g), lambda i: (0, 0)),
            pl.BlockSpec((k, g), lambda i: (0, 0)),
            pl.BlockSpec((g, num_out), lambda i: (0, 0)),
        ],
        out_specs=pl.BlockSpec((_BN, num_out), lambda i: (i, 0)),
        out_shape=jax.ShapeDtypeStruct((n, num_out), jnp.float32),
    )(x, pos_t, scl_t, rot_t, coefs, freqs, gaussian_colors)
    return out
